# Initial kernel scaffold; baseline (speedup 1.0000x reference)
import jax
import jax.numpy as jnp
from jax.experimental import pallas as pl


def kernel(x, edge_index, edge_attr, W_init, W_edge, b_edge, eps, att_vec, W_u1, b_u1, W_u2, b_u2, W_o1, b_o1, bn_gamma, bn_beta, W_o2, b_o2):
    raise NotImplementedError("write your pallas kernel here")


# speedup vs baseline: 5.6416x; 5.6416x over previous
import functools

import jax
import jax.numpy as jnp
from jax import lax
from jax.experimental import pallas as pl
from jax.experimental.pallas import tpu as pltpu
from jax.experimental.pallas import tpu_sc as plsc

N = 10000
E = 320000
D = 128
DE = 16
HID = 128

NC = 2
NS = 16
NW = NC * NS
C = 128
CPW = 79
EPW = CPW * C
E_PAD = NW * EPW
LAST_REAL = (E - (NW - 1) * EPW) // C
NP = 10240

f32 = jnp.float32


@functools.lru_cache(maxsize=None)
def _mesh():
    return plsc.VectorSubcoreMesh(core_axis_name="c", subcore_axis_name="s",
                                  num_cores=NC, num_subcores=NS)


def _wid_base():
    c = lax.axis_index("c")
    s = lax.axis_index("s")
    wid = s * NC + c
    return c, s, wid, wid * EPW


def _nchunks(wid):
    return jnp.where(wid == NW - 1, LAST_REAL, CPW)


def _gather_body(h_hbm, srcp_hbm, hsrc_hbm, idx_v, buf, sem):
    _, _, wid, base = _wid_base()
    pltpu.sync_copy(srcp_hbm.at[pl.ds(wid * EPW, EPW)], idx_v)

    def body(j, carry):
        pltpu.async_copy(h_hbm.at[idx_v.at[pl.ds(j * C, C)]], buf, sem).wait()
        pltpu.sync_copy(buf, hsrc_hbm.at[pl.ds(base + j * C, C)])
        return carry

    lax.fori_loop(0, _nchunks(wid), body, 0)


@functools.lru_cache(maxsize=None)
def _gather():
    return pl.kernel(
        _gather_body,
        out_type=jax.ShapeDtypeStruct((E, D), f32),
        mesh=_mesh(),
        scratch_types=[
            pltpu.VMEM((EPW,), jnp.int32),
            pltpu.VMEM((C, D), f32),
            pltpu.SemaphoreType.DMA,
        ],
    )


_AS = 632
_DS = NP // NS


def _scatter_body(wmsg_hbm, p_hbm, dstp_hbm, zeros_hbm,
                  aggpart_hbm, denpart_hbm,
                  agg_sh, den_sh, idx_v, buf, pbuf, den_v, dbuf, dsum):
    c, s, wid, base = _wid_base()

    def zden(i, carry):
        den_v[pl.ds(i * 16, 16)] = jnp.zeros((16,), f32)
        return carry

    lax.fori_loop(0, NP // 16, zden, 0)

    def zagg(sz):
        pltpu.sync_copy(zeros_hbm.at[pl.ds(s * _AS, sz)],
                        agg_sh.at[pl.ds(s * _AS, sz)])

    @pl.when(s < NS - 1)
    def _():
        zagg(_AS)

    @pl.when(s == NS - 1)
    def _():
        zagg(N - (NS - 1) * _AS)

    plsc.subcore_barrier()

    pltpu.sync_copy(dstp_hbm.at[pl.ds(wid * EPW, EPW)], idx_v)

    def body(j, carry):
        pltpu.sync_copy(wmsg_hbm.at[pl.ds(base + j * C, C)], buf)
        pltpu.sync_copy(buf, agg_sh.at[idx_v.at[pl.ds(j * C, C)]], add=True)
        pltpu.sync_copy(p_hbm.at[pl.ds(base + j * C, C)], pbuf)
        for k in range(C // 16):
            idxk = idx_v[pl.ds(j * C + k * 16, 16)]
            pk = pbuf[pl.ds(k * 16, 16)]
            plsc.addupdate_scatter(den_v, [idxk], pk)
        return carry

    lax.fori_loop(0, _nchunks(wid), body, 0)
    plsc.subcore_barrier()

    def wagg(sz):
        pltpu.sync_copy(agg_sh.at[pl.ds(s * _AS, sz)],
                        aggpart_hbm.at[c, pl.ds(s * _AS, sz)])

    @pl.when(s < NS - 1)
    def _():
        wagg(_AS)

    @pl.when(s == NS - 1)
    def _():
        wagg(N - (NS - 1) * _AS)

    pltpu.sync_copy(den_v, den_sh.at[pl.ds(s * NP, NP)])
    plsc.subcore_barrier()
    off = s * _DS
    pltpu.sync_copy(den_sh.at[pl.ds(off, _DS)], dsum)
    for k in range(1, NS):
        pltpu.sync_copy(den_sh.at[pl.ds(k * NP + off, _DS)], dbuf)
        for g in range(_DS // 16):
            sl = pl.ds(g * 16, 16)
            dsum[sl] = dsum[sl] + dbuf[sl]
    pltpu.sync_copy(dsum, denpart_hbm.at[pl.ds(c * NP + off, _DS)])


@functools.lru_cache(maxsize=None)
def _scatter():
    return pl.kernel(
        _scatter_body,
        out_type=(jax.ShapeDtypeStruct((NC, N, D), f32),
                  jax.ShapeDtypeStruct((NC * NP,), f32)),
        mesh=_mesh(),
        scratch_types=[
            pltpu.VMEM_SHARED((N, D), f32),
            pltpu.VMEM_SHARED((NS * NP,), f32),
            pltpu.VMEM((EPW,), jnp.int32),
            pltpu.VMEM((C, D), f32),
            pltpu.VMEM((C,), f32),
            pltpu.VMEM((NP,), f32),
            pltpu.VMEM((_DS,), f32),
            pltpu.VMEM((_DS,), f32),
        ],
        compiler_params=pltpu.CompilerParams(needs_layout_passes=False),
    )


def _attn_body(p_hbm, dstp_hbm, den0_hbm, den1_hbm, attn_hbm,
               den_v, dbb, idx_v, pbuf, abuf):
    _, _, wid, base = _wid_base()
    pltpu.sync_copy(den0_hbm, den_v)
    pltpu.sync_copy(den1_hbm, dbb)

    def addden(i, carry):
        sl = pl.ds(i * 16, 16)
        den_v[sl] = den_v[sl] + dbb[sl] + 1e-16
        return carry

    lax.fori_loop(0, NP // 16, addden, 0)
    pltpu.sync_copy(dstp_hbm.at[pl.ds(wid * EPW, EPW)], idx_v)

    def body(j, carry):
        pltpu.sync_copy(p_hbm.at[pl.ds(base + j * C, C)], pbuf)
        for k in range(C // 16):
            idxk = idx_v[pl.ds(j * C + k * 16, 16)]
            d = plsc.load_gather(den_v, [idxk])
            abuf[pl.ds(k * 16, 16)] = pbuf[pl.ds(k * 16, 16)] / d
        pltpu.sync_copy(abuf, attn_hbm.at[pl.ds(base + j * C, C)])
        return carry

    lax.fori_loop(0, _nchunks(wid), body, 0)


@functools.lru_cache(maxsize=None)
def _attn():
    return pl.kernel(
        _attn_body,
        out_type=jax.ShapeDtypeStruct((E,), f32),
        mesh=_mesh(),
        scratch_types=[
            pltpu.VMEM((NP,), f32),
            pltpu.VMEM((NP,), f32),
            pltpu.VMEM((EPW,), jnp.int32),
            pltpu.VMEM((C,), f32),
            pltpu.VMEM((C,), f32),
        ],
        compiler_params=pltpu.CompilerParams(needs_layout_passes=False),
    )


def _k1_body(x_ref, w_ref, h_ref):
    h_ref[...] = jnp.dot(x_ref[...], w_ref[...], preferred_element_type=f32)


def _k1(x, w_t):
    nb = 1000
    return pl.pallas_call(
        _k1_body,
        grid=(N // nb,),
        in_specs=[pl.BlockSpec((nb, D), lambda i: (i, 0)),
                  pl.BlockSpec((D, D), lambda i: (0, 0))],
        out_specs=pl.BlockSpec((nb, D), lambda i: (i, 0)),
        out_shape=jax.ShapeDtypeStruct((N, D), f32),
    )(x, w_t)


_EB = 512


def _k3_body(hsrc_ref, ea_ref, wet_ref, be_ref, av_ref, wmsg_ref, p_ref):
    ea = ea_ref[...]
    nrm = jnp.sqrt(jnp.sum(ea * ea, axis=1, keepdims=True)) + 1e-8
    ep = jnp.dot(ea / nrm, wet_ref[...], preferred_element_type=f32) + be_ref[...]
    msgs = jnp.tanh(hsrc_ref[...] + ep)
    att = jnp.sum(msgs * av_ref[...], axis=1, keepdims=True)
    p = jnp.exp(att)
    wmsg_ref[...] = msgs * p
    p_ref[...] = p.reshape(1, _EB, 1)


def _k3(hsrc, ea, wet, be, av):
    nblk = E // _EB
    return pl.pallas_call(
        _k3_body,
        grid=(nblk,),
        in_specs=[pl.BlockSpec((_EB, D), lambda i: (i, 0)),
                  pl.BlockSpec((_EB, DE), lambda i: (i, 0)),
                  pl.BlockSpec((DE, D), lambda i: (0, 0)),
                  pl.BlockSpec((1, D), lambda i: (0, 0)),
                  pl.BlockSpec((1, D), lambda i: (0, 0))],
        out_specs=[pl.BlockSpec((_EB, D), lambda i: (i, 0)),
                   pl.BlockSpec((1, _EB, 1), lambda i: (i, 0, 0))],
        out_shape=(jax.ShapeDtypeStruct((E, D), f32),
                   jax.ShapeDtypeStruct((nblk, _EB, 1), f32)),
    )(hsrc, ea, wet, be, av)


_NB = 1000


def _k5a_body(h_ref, agg_ref, den0_ref, den1_ref, eps_ref, wu1_ref, bu1_ref,
              wu2_ref, bu2_ref, wo1_ref, bo1_ref,
              o1_ref, csum_ref, csq_ref):
    i = pl.program_id(0)
    den = (den0_ref[...].reshape(_NB, 1) + den1_ref[...].reshape(_NB, 1)
           + 1e-16)
    agg = (agg_ref[0] + agg_ref[1]) / den
    pre = (1.0 + eps_ref[0, 0]) * h_ref[...] + agg
    t = jnp.tanh(jnp.dot(pre, wu1_ref[...], preferred_element_type=f32)
                 + bu1_ref[...])
    u = jnp.dot(t, wu2_ref[...], preferred_element_type=f32) + bu2_ref[...]
    o1 = jnp.dot(u, wo1_ref[...], preferred_element_type=f32) + bo1_ref[...]
    o1_ref[...] = o1
    s = jnp.sum(o1, axis=0, keepdims=True)
    s2 = jnp.sum(o1 * o1, axis=0, keepdims=True)

    @pl.when(i == 0)
    def _():
        csum_ref[...] = s
        csq_ref[...] = s2

    @pl.when(i > 0)
    def _():
        csum_ref[...] = csum_ref[...] + s
        csq_ref[...] = csq_ref[...] + s2


def _k5a(h, aggpart, den0, den1, eps2, wu1t, bu1, wu2t, bu2, wo1t, bo1):
    nblk = N // _NB
    return pl.pallas_call(
        _k5a_body,
        grid=(nblk,),
        in_specs=[pl.BlockSpec((_NB, D), lambda i: (i, 0)),
                  pl.BlockSpec((NC, _NB, D), lambda i: (0, i, 0)),
                  pl.BlockSpec((1, _NB, 1), lambda i: (i, 0, 0)),
                  pl.BlockSpec((1, _NB, 1), lambda i: (i, 0, 0)),
                  pl.BlockSpec((1, 1), lambda i: (0, 0)),
                  pl.BlockSpec((D, HID), lambda i: (0, 0)),
                  pl.BlockSpec((1, HID), lambda i: (0, 0)),
                  pl.BlockSpec((HID, D), lambda i: (0, 0)),
                  pl.BlockSpec((1, D), lambda i: (0, 0)),
                  pl.BlockSpec((D, HID), lambda i: (0, 0)),
                  pl.BlockSpec((1, HID), lambda i: (0, 0))],
        out_specs=[pl.BlockSpec((_NB, HID), lambda i: (i, 0)),
                   pl.BlockSpec((1, HID), lambda i: (0, 0)),
                   pl.BlockSpec((1, HID), lambda i: (0, 0))],
        out_shape=(jax.ShapeDtypeStruct((N, HID), f32),
                   jax.ShapeDtypeStruct((1, HID), f32),
                   jax.ShapeDtypeStruct((1, HID), f32)),
    )(h, aggpart, den0, den1, eps2, wu1t, bu1, wu2t, bu2, wo1t, bo1)


def _k5b_body(o1_ref, csum_ref, csq_ref, g_ref, b_ref, wo2_ref, bo2_ref,
              out_ref):
    mean = csum_ref[...] * (1.0 / N)
    var = csq_ref[...] * (1.0 / N) - mean * mean
    rstd = 1.0 / jnp.sqrt(var + 1e-5)
    o1n = (o1_ref[...] - mean) * rstd * g_ref[...] + b_ref[...]
    out_ref[...] = (jnp.dot(jnp.tanh(o1n), wo2_ref[...],
                            preferred_element_type=f32) + bo2_ref[...])


def _k5b(o1, csum, csq, g, b, wo2t, bo2):
    nblk = N // _NB
    return pl.pallas_call(
        _k5b_body,
        grid=(nblk,),
        in_specs=[pl.BlockSpec((_NB, HID), lambda i: (i, 0)),
                  pl.BlockSpec((1, HID), lambda i: (0, 0)),
                  pl.BlockSpec((1, HID), lambda i: (0, 0)),
                  pl.BlockSpec((1, HID), lambda i: (0, 0)),
                  pl.BlockSpec((1, HID), lambda i: (0, 0)),
                  pl.BlockSpec((HID, D), lambda i: (0, 0)),
                  pl.BlockSpec((1, D), lambda i: (0, 0))],
        out_specs=pl.BlockSpec((_NB, D), lambda i: (i, 0)),
        out_shape=jax.ShapeDtypeStruct((N, D), f32),
    )(o1, csum, csq, g, b, wo2t, bo2)


def kernel(x, edge_index, edge_attr, W_init, W_edge, b_edge, eps, att_vec,
           W_u1, b_u1, W_u2, b_u2, W_o1, b_o1, bn_gamma, bn_beta, W_o2, b_o2):
    src = edge_index[0]
    dst = edge_index[1]
    srcp = jnp.pad(src, (0, E_PAD - E))
    dstf = jnp.pad(dst, (0, E_PAD - E))
    zeros = jnp.zeros((N, D), f32)

    h = _k1(x, W_init.T)
    hsrc = _gather()(h, srcp)
    wmsg, p3 = _k3(hsrc, edge_attr, W_edge.T, b_edge.reshape(1, D),
                   att_vec.reshape(1, D))
    p = p3.reshape(E)
    aggpart, denpart = _scatter()(wmsg, p, dstf, zeros)
    dp = denpart.reshape(NC, NP)
    den0 = dp[0, :N].reshape(N // _NB, _NB, 1)
    den1 = dp[1, :N].reshape(N // _NB, _NB, 1)
    o1, csum, csq = _k5a(h, aggpart, den0, den1, eps.reshape(1, 1),
                         W_u1.T, b_u1.reshape(1, HID),
                         W_u2.T, b_u2.reshape(1, D),
                         W_o1.T, b_o1.reshape(1, HID))
    out = _k5b(o1, csum, csq, bn_gamma.reshape(1, HID),
               bn_beta.reshape(1, HID), W_o2.T, b_o2.reshape(1, D))
    attn = _attn()(p, dstf, dp[0], dp[1])
    return out, attn


# speedup vs baseline: 6.1542x; 1.0908x over previous
import functools

import jax
import jax.numpy as jnp
from jax import lax
from jax.experimental import pallas as pl
from jax.experimental.pallas import tpu as pltpu
from jax.experimental.pallas import tpu_sc as plsc

N = 10000
E = 320000
D = 128
DE = 16
HID = 128

NC = 2
NS = 16
NW = NC * NS
C = 128
CPW = 79
EPW = CPW * C
E_PAD = NW * EPW
LAST_REAL = (E - (NW - 1) * EPW) // C
NP = 10240

f32 = jnp.float32


@functools.lru_cache(maxsize=None)
def _mesh():
    return plsc.VectorSubcoreMesh(core_axis_name="c", subcore_axis_name="s",
                                  num_cores=NC, num_subcores=NS)


def _wid_base():
    c = lax.axis_index("c")
    s = lax.axis_index("s")
    wid = s * NC + c
    return c, s, wid, wid * EPW


def _nchunks(wid):
    return jnp.where(wid == NW - 1, LAST_REAL, CPW)


def _gather_body(h_hbm, srcp_hbm, hsrc_hbm, idx_v, buf, sem):
    _, _, wid, base = _wid_base()
    pltpu.sync_copy(srcp_hbm.at[pl.ds(wid * EPW, EPW)], idx_v)

    def body(j, carry):
        pltpu.async_copy(h_hbm.at[idx_v.at[pl.ds(j * C, C)]], buf, sem).wait()
        pltpu.sync_copy(buf, hsrc_hbm.at[pl.ds(base + j * C, C)])
        return carry

    lax.fori_loop(0, _nchunks(wid), body, 0)


@functools.lru_cache(maxsize=None)
def _gather():
    return pl.kernel(
        _gather_body,
        out_type=jax.ShapeDtypeStruct((E, D), f32),
        mesh=_mesh(),
        scratch_types=[
            pltpu.VMEM((EPW,), jnp.int32),
            pltpu.VMEM((C, D), f32),
            pltpu.SemaphoreType.DMA,
        ],
    )


_AS = 632
_DS = NP // NS


def _scatter_body(wmsg_hbm, p_hbm, dstp_hbm, zeros_hbm,
                  aggpart_hbm, denpart_hbm,
                  agg_sh, den_sh, idx_v, buf, pbuf, den_v, dbuf, dsum):
    c, s, wid, base = _wid_base()

    def zden(i, carry):
        den_v[pl.ds(i * 16, 16)] = jnp.zeros((16,), f32)
        return carry

    lax.fori_loop(0, NP // 16, zden, 0)

    def zagg(sz):
        pltpu.sync_copy(zeros_hbm.at[pl.ds(s * _AS, sz)],
                        agg_sh.at[pl.ds(s * _AS, sz)])

    @pl.when(s < NS - 1)
    def _():
        zagg(_AS)

    @pl.when(s == NS - 1)
    def _():
        zagg(N - (NS - 1) * _AS)

    plsc.subcore_barrier()

    pltpu.sync_copy(dstp_hbm.at[pl.ds(wid * EPW, EPW)], idx_v)

    def body(j, carry):
        pltpu.sync_copy(wmsg_hbm.at[pl.ds(base + j * C, C)], buf)
        pltpu.sync_copy(buf, agg_sh.at[idx_v.at[pl.ds(j * C, C)]], add=True)
        pltpu.sync_copy(p_hbm.at[pl.ds(base + j * C, C)], pbuf)
        for k in range(C // 16):
            idxk = idx_v[pl.ds(j * C + k * 16, 16)]
            pk = pbuf[pl.ds(k * 16, 16)]
            plsc.addupdate_scatter(den_v, [idxk], pk)
        return carry

    lax.fori_loop(0, _nchunks(wid), body, 0)
    plsc.subcore_barrier()

    def wagg(sz):
        pltpu.sync_copy(agg_sh.at[pl.ds(s * _AS, sz)],
                        aggpart_hbm.at[c, pl.ds(s * _AS, sz)])

    @pl.when(s < NS - 1)
    def _():
        wagg(_AS)

    @pl.when(s == NS - 1)
    def _():
        wagg(N - (NS - 1) * _AS)

    pltpu.sync_copy(den_v, den_sh.at[pl.ds(s * NP, NP)])
    plsc.subcore_barrier()
    off = s * _DS
    pltpu.sync_copy(den_sh.at[pl.ds(off, _DS)], dsum)
    for k in range(1, NS):
        pltpu.sync_copy(den_sh.at[pl.ds(k * NP + off, _DS)], dbuf)
        for g in range(_DS // 16):
            sl = pl.ds(g * 16, 16)
            dsum[sl] = dsum[sl] + dbuf[sl]
    pltpu.sync_copy(dsum, denpart_hbm.at[pl.ds(c * NP + off, _DS)])


@functools.lru_cache(maxsize=None)
def _scatter():
    return pl.kernel(
        _scatter_body,
        out_type=(jax.ShapeDtypeStruct((NC, N, D), f32),
                  jax.ShapeDtypeStruct((NC * NP,), f32)),
        mesh=_mesh(),
        scratch_types=[
            pltpu.VMEM_SHARED((N, D), f32),
            pltpu.VMEM_SHARED((NS * NP,), f32),
            pltpu.VMEM((EPW,), jnp.int32),
            pltpu.VMEM((C, D), f32),
            pltpu.VMEM((C,), f32),
            pltpu.VMEM((NP,), f32),
            pltpu.VMEM((_DS,), f32),
            pltpu.VMEM((_DS,), f32),
        ],
        compiler_params=pltpu.CompilerParams(needs_layout_passes=False),
    )


def _attn_body(p_hbm, dstp_hbm, den0_hbm, den1_hbm, attn_hbm,
               den_v, dbb, idx_v, pbuf, abuf):
    _, _, wid, base = _wid_base()
    pltpu.sync_copy(den0_hbm, den_v)
    pltpu.sync_copy(den1_hbm, dbb)

    def addden(i, carry):
        sl = pl.ds(i * 16, 16)
        den_v[sl] = den_v[sl] + dbb[sl] + 1e-16
        return carry

    lax.fori_loop(0, NP // 16, addden, 0)
    pltpu.sync_copy(dstp_hbm.at[pl.ds(wid * EPW, EPW)], idx_v)

    def body(j, carry):
        pltpu.sync_copy(p_hbm.at[pl.ds(base + j * C, C)], pbuf)
        for k in range(C // 16):
            idxk = idx_v[pl.ds(j * C + k * 16, 16)]
            d = plsc.load_gather(den_v, [idxk])
            abuf[pl.ds(k * 16, 16)] = pbuf[pl.ds(k * 16, 16)] / d
        pltpu.sync_copy(abuf, attn_hbm.at[pl.ds(base + j * C, C)])
        return carry

    lax.fori_loop(0, _nchunks(wid), body, 0)


@functools.lru_cache(maxsize=None)
def _attn():
    return pl.kernel(
        _attn_body,
        out_type=jax.ShapeDtypeStruct((E,), f32),
        mesh=_mesh(),
        scratch_types=[
            pltpu.VMEM((NP,), f32),
            pltpu.VMEM((NP,), f32),
            pltpu.VMEM((EPW,), jnp.int32),
            pltpu.VMEM((C,), f32),
            pltpu.VMEM((C,), f32),
        ],
        compiler_params=pltpu.CompilerParams(needs_layout_passes=False),
    )


def _k1_body(x_ref, w_ref, h_ref):
    h_ref[...] = jnp.dot(x_ref[...], w_ref[...], preferred_element_type=f32)


def _k1(x, w_t):
    nb = 1000
    return pl.pallas_call(
        _k1_body,
        grid=(N // nb,),
        in_specs=[pl.BlockSpec((nb, D), lambda i: (i, 0)),
                  pl.BlockSpec((D, D), lambda i: (0, 0))],
        out_specs=pl.BlockSpec((nb, D), lambda i: (i, 0)),
        out_shape=jax.ShapeDtypeStruct((N, D), f32),
    )(x, w_t)


_EB = 512


def _k3_body(hsrc_ref, eat_ref, we_ref, be_ref, av_ref, wmsg_ref, p_ref):
    eat = eat_ref[...]
    nrm = jnp.sqrt(jnp.sum(eat * eat, axis=0, keepdims=True)) + 1e-8
    ean_t = eat / nrm
    ep = lax.dot_general(ean_t, we_ref[...], (((0,), (1,)), ((), ())),
                         preferred_element_type=f32) + be_ref[...]
    msgs = jnp.tanh(hsrc_ref[...] + ep)
    att = jnp.dot(msgs, av_ref[...], preferred_element_type=f32)
    p = jnp.exp(att)
    wmsg_ref[...] = msgs * p
    att_row = lax.dot_general(av_ref[...], msgs, (((0,), (1,)), ((), ())),
                              preferred_element_type=f32)
    p_ref[...] = jnp.exp(att_row).reshape(1, 1, _EB)


def _k3(hsrc, eat, we, be, av):
    nblk = E // _EB
    return pl.pallas_call(
        _k3_body,
        grid=(nblk,),
        in_specs=[pl.BlockSpec((_EB, D), lambda i: (i, 0)),
                  pl.BlockSpec((DE, _EB), lambda i: (0, i)),
                  pl.BlockSpec((D, DE), lambda i: (0, 0)),
                  pl.BlockSpec((1, D), lambda i: (0, 0)),
                  pl.BlockSpec((D, 1), lambda i: (0, 0))],
        out_specs=[pl.BlockSpec((_EB, D), lambda i: (i, 0)),
                   pl.BlockSpec((1, 1, _EB), lambda i: (i, 0, 0))],
        out_shape=(jax.ShapeDtypeStruct((E, D), f32),
                   jax.ShapeDtypeStruct((nblk, 1, _EB), f32)),
    )(hsrc, eat, we, be, av)


_NB = 1000


def _k5a_body(h_ref, agg_ref, den0_ref, den1_ref, eps_ref, wu1_ref, bu1_ref,
              wu2_ref, bu2_ref, wo1_ref, bo1_ref,
              o1_ref, csum_ref, csq_ref):
    i = pl.program_id(0)
    den = (den0_ref[...].reshape(_NB, 1) + den1_ref[...].reshape(_NB, 1)
           + 1e-16)
    agg = (agg_ref[0] + agg_ref[1]) / den
    pre = (1.0 + eps_ref[0, 0]) * h_ref[...] + agg
    t = jnp.tanh(jnp.dot(pre, wu1_ref[...], preferred_element_type=f32)
                 + bu1_ref[...])
    u = jnp.dot(t, wu2_ref[...], preferred_element_type=f32) + bu2_ref[...]
    o1 = jnp.dot(u, wo1_ref[...], preferred_element_type=f32) + bo1_ref[...]
    o1_ref[...] = o1
    s = jnp.sum(o1, axis=0, keepdims=True)
    s2 = jnp.sum(o1 * o1, axis=0, keepdims=True)

    @pl.when(i == 0)
    def _():
        csum_ref[...] = s
        csq_ref[...] = s2

    @pl.when(i > 0)
    def _():
        csum_ref[...] = csum_ref[...] + s
        csq_ref[...] = csq_ref[...] + s2


def _k5a(h, aggpart, den0, den1, eps2, wu1t, bu1, wu2t, bu2, wo1t, bo1):
    nblk = N // _NB
    return pl.pallas_call(
        _k5a_body,
        grid=(nblk,),
        in_specs=[pl.BlockSpec((_NB, D), lambda i: (i, 0)),
                  pl.BlockSpec((NC, _NB, D), lambda i: (0, i, 0)),
                  pl.BlockSpec((1, _NB, 1), lambda i: (i, 0, 0)),
                  pl.BlockSpec((1, _NB, 1), lambda i: (i, 0, 0)),
                  pl.BlockSpec((1, 1), lambda i: (0, 0)),
                  pl.BlockSpec((D, HID), lambda i: (0, 0)),
                  pl.BlockSpec((1, HID), lambda i: (0, 0)),
                  pl.BlockSpec((HID, D), lambda i: (0, 0)),
                  pl.BlockSpec((1, D), lambda i: (0, 0)),
                  pl.BlockSpec((D, HID), lambda i: (0, 0)),
                  pl.BlockSpec((1, HID), lambda i: (0, 0))],
        out_specs=[pl.BlockSpec((_NB, HID), lambda i: (i, 0)),
                   pl.BlockSpec((1, HID), lambda i: (0, 0)),
                   pl.BlockSpec((1, HID), lambda i: (0, 0))],
        out_shape=(jax.ShapeDtypeStruct((N, HID), f32),
                   jax.ShapeDtypeStruct((1, HID), f32),
                   jax.ShapeDtypeStruct((1, HID), f32)),
    )(h, aggpart, den0, den1, eps2, wu1t, bu1, wu2t, bu2, wo1t, bo1)


def _k5b_body(o1_ref, csum_ref, csq_ref, g_ref, b_ref, wo2_ref, bo2_ref,
              out_ref):
    mean = csum_ref[...] * (1.0 / N)
    var = csq_ref[...] * (1.0 / N) - mean * mean
    rstd = 1.0 / jnp.sqrt(var + 1e-5)
    o1n = (o1_ref[...] - mean) * rstd * g_ref[...] + b_ref[...]
    out_ref[...] = (jnp.dot(jnp.tanh(o1n), wo2_ref[...],
                            preferred_element_type=f32) + bo2_ref[...])


def _k5b(o1, csum, csq, g, b, wo2t, bo2):
    nblk = N // _NB
    return pl.pallas_call(
        _k5b_body,
        grid=(nblk,),
        in_specs=[pl.BlockSpec((_NB, HID), lambda i: (i, 0)),
                  pl.BlockSpec((1, HID), lambda i: (0, 0)),
                  pl.BlockSpec((1, HID), lambda i: (0, 0)),
                  pl.BlockSpec((1, HID), lambda i: (0, 0)),
                  pl.BlockSpec((1, HID), lambda i: (0, 0)),
                  pl.BlockSpec((HID, D), lambda i: (0, 0)),
                  pl.BlockSpec((1, D), lambda i: (0, 0))],
        out_specs=pl.BlockSpec((_NB, D), lambda i: (i, 0)),
        out_shape=jax.ShapeDtypeStruct((N, D), f32),
    )(o1, csum, csq, g, b, wo2t, bo2)


def kernel(x, edge_index, edge_attr, W_init, W_edge, b_edge, eps, att_vec,
           W_u1, b_u1, W_u2, b_u2, W_o1, b_o1, bn_gamma, bn_beta, W_o2, b_o2):
    src = edge_index[0]
    dst = edge_index[1]
    srcp = jnp.pad(src, (0, E_PAD - E))
    dstf = jnp.pad(dst, (0, E_PAD - E))
    zeros = jnp.zeros((N, D), f32)

    h = _k1(x, W_init.T)
    hsrc = _gather()(h, srcp)
    wmsg, p3 = _k3(hsrc, edge_attr.T, W_edge, b_edge.reshape(1, D), att_vec)
    p = p3.reshape(E)
    aggpart, denpart = _scatter()(wmsg, p, dstf, zeros)
    dp = denpart.reshape(NC, NP)
    den0 = dp[0, :N].reshape(N // _NB, _NB, 1)
    den1 = dp[1, :N].reshape(N // _NB, _NB, 1)
    o1, csum, csq = _k5a(h, aggpart, den0, den1, eps.reshape(1, 1),
                         W_u1.T, b_u1.reshape(1, HID),
                         W_u2.T, b_u2.reshape(1, D),
                         W_o1.T, b_o1.reshape(1, HID))
    out = _k5b(o1, csum, csq, bn_gamma.reshape(1, HID),
               bn_beta.reshape(1, HID), W_o2.T, b_o2.reshape(1, D))
    attn = _attn()(p, dstf, dp[0], dp[1])
    return out, attn


# speedup vs baseline: 7.3233x; 1.1900x over previous
import functools

import jax
import jax.numpy as jnp
from jax import lax
from jax.experimental import pallas as pl
from jax.experimental.pallas import tpu as pltpu
from jax.experimental.pallas import tpu_sc as plsc

N = 10000
E = 320000
D = 128
DE = 16
HID = 128

NC = 2
NS = 16
NW = NC * NS
C = 128
CPW = 79
EPW = CPW * C
E_PAD = NW * EPW
LAST_REAL = (E - (NW - 1) * EPW) // C
NP = 10240

f32 = jnp.float32


@functools.lru_cache(maxsize=None)
def _mesh():
    return plsc.VectorSubcoreMesh(core_axis_name="c", subcore_axis_name="s",
                                  num_cores=NC, num_subcores=NS)


def _wid_base():
    c = lax.axis_index("c")
    s = lax.axis_index("s")
    wid = s * NC + c
    return c, s, wid, wid * EPW


def _nchunks(wid):
    return jnp.where(wid == NW - 1, LAST_REAL, CPW)


def _gather_body(h_hbm, srcp_hbm, hsrc_hbm, idx_v, buf, sem0, sem1):
    _, _, wid, base = _wid_base()
    pltpu.sync_copy(srcp_hbm.at[pl.ds(wid * EPW, EPW)], idx_v)
    n = _nchunks(wid)
    sems = (sem0, sem1)

    def start(j, slot):
        pltpu.async_copy(h_hbm.at[idx_v.at[pl.ds(j * C, C)]],
                         buf.at[slot], sems[slot])

    def finish(j, slot):
        pltpu.make_async_copy(h_hbm.at[pl.ds(0, C)], buf.at[slot],
                              sems[slot]).wait()
        pltpu.sync_copy(buf.at[slot], hsrc_hbm.at[pl.ds(base + j * C, C)])

    start(0, 0)

    def body(i, carry):
        j0 = 2 * i
        j1 = j0 + 1
        start(j1, 1)
        finish(j0, 0)

        @pl.when(j1 + 1 < n)
        def _():
            start(j1 + 1, 0)

        finish(j1, 1)
        return carry

    lax.fori_loop(0, (n - 1) // 2, body, 0)
    finish(n - 1, 0)


@functools.lru_cache(maxsize=None)
def _gather():
    return pl.kernel(
        _gather_body,
        out_type=jax.ShapeDtypeStruct((E, D), f32),
        mesh=_mesh(),
        scratch_types=[
            pltpu.VMEM((EPW,), jnp.int32),
            pltpu.VMEM((2, C, D), f32),
            pltpu.SemaphoreType.DMA,
            pltpu.SemaphoreType.DMA,
        ],
    )


_AS = 632
_DS = NP // NS


def _scatter_body(wmsg_hbm, p_hbm, dstp_hbm, zeros_hbm, zf_hbm,
                  aggpart_hbm, denpart_hbm,
                  agg_sh, den_sh, idx_v, buf0, buf1, pbuf0, pbuf1,
                  semw0, semw1, semp0, semp1):
    c, s, wid, base = _wid_base()
    semw = (semw0, semw1)
    semp = (semp0, semp1)
    bufs = (buf0, buf1)
    pbufs = (pbuf0, pbuf1)

    def zagg(sz):
        pltpu.sync_copy(zeros_hbm.at[pl.ds(s * _AS, sz)],
                        agg_sh.at[pl.ds(s * _AS, sz)])

    @pl.when(s < NS - 1)
    def _():
        zagg(_AS)

    @pl.when(s == NS - 1)
    def _():
        zagg(N - (NS - 1) * _AS)

    pltpu.sync_copy(zf_hbm.at[pl.ds(s * _DS, _DS)],
                    den_sh.at[pl.ds(s * _DS, _DS)])
    plsc.subcore_barrier()

    pltpu.sync_copy(dstp_hbm.at[pl.ds(wid * EPW, EPW)], idx_v)
    n = _nchunks(wid)

    def start(j, slot):
        pltpu.async_copy(wmsg_hbm.at[pl.ds(base + j * C, C)],
                         bufs[slot], semw[slot])
        pltpu.async_copy(p_hbm.at[pl.ds(base + j * C, C)],
                         pbufs[slot], semp[slot])

    def finish(j, slot):
        pltpu.make_async_copy(wmsg_hbm.at[pl.ds(0, C)], bufs[slot],
                              semw[slot]).wait()
        pltpu.make_async_copy(p_hbm.at[pl.ds(0, C)], pbufs[slot],
                              semp[slot]).wait()
        pltpu.sync_copy(bufs[slot],
                        agg_sh.at[idx_v.at[pl.ds(j * C, C)]], add=True)
        pltpu.sync_copy(pbufs[slot],
                        den_sh.at[idx_v.at[pl.ds(j * C, C)]], add=True)

    start(0, 0)

    def body(i, carry):
        j0 = 2 * i
        j1 = j0 + 1
        start(j1, 1)
        finish(j0, 0)

        @pl.when(j1 + 1 < n)
        def _():
            start(j1 + 1, 0)

        finish(j1, 1)
        return carry

    lax.fori_loop(0, (n - 1) // 2, body, 0)
    finish(n - 1, 0)
    plsc.subcore_barrier()

    def wagg(sz):
        pltpu.sync_copy(agg_sh.at[pl.ds(s * _AS, sz)],
                        aggpart_hbm.at[c, pl.ds(s * _AS, sz)])

    @pl.when(s < NS - 1)
    def _():
        wagg(_AS)

    @pl.when(s == NS - 1)
    def _():
        wagg(N - (NS - 1) * _AS)

    pltpu.sync_copy(den_sh.at[pl.ds(s * _DS, _DS)],
                    denpart_hbm.at[pl.ds(c * NP + s * _DS, _DS)])


@functools.lru_cache(maxsize=None)
def _scatter():
    return pl.kernel(
        _scatter_body,
        out_type=(jax.ShapeDtypeStruct((NC, N, D), f32),
                  jax.ShapeDtypeStruct((NC * NP,), f32)),
        mesh=_mesh(),
        scratch_types=[
            pltpu.VMEM_SHARED((N, D), f32),
            pltpu.VMEM_SHARED((NP,), f32),
            pltpu.VMEM((EPW,), jnp.int32),
            pltpu.VMEM((C, D), f32),
            pltpu.VMEM((C, D), f32),
            pltpu.VMEM((C,), f32),
            pltpu.VMEM((C,), f32),
            pltpu.SemaphoreType.DMA,
            pltpu.SemaphoreType.DMA,
            pltpu.SemaphoreType.DMA,
            pltpu.SemaphoreType.DMA,
        ],
    )


def _attn_body(p_hbm, dstp_hbm, den0_hbm, den1_hbm, attn_hbm,
               den_v, dbb, idx_v, pbuf, abuf):
    _, _, wid, base = _wid_base()
    pltpu.sync_copy(den0_hbm, den_v)
    pltpu.sync_copy(den1_hbm, dbb)

    def addden(i, carry):
        sl = pl.ds(i * 16, 16)
        den_v[sl] = den_v[sl] + dbb[sl] + 1e-16
        return carry

    lax.fori_loop(0, NP // 16, addden, 0)
    pltpu.sync_copy(dstp_hbm.at[pl.ds(wid * EPW, EPW)], idx_v)

    def body(j, carry):
        pltpu.sync_copy(p_hbm.at[pl.ds(base + j * C, C)], pbuf)
        for k in range(C // 16):
            idxk = idx_v[pl.ds(j * C + k * 16, 16)]
            d = plsc.load_gather(den_v, [idxk])
            abuf[pl.ds(k * 16, 16)] = pbuf[pl.ds(k * 16, 16)] / d
        pltpu.sync_copy(abuf, attn_hbm.at[pl.ds(base + j * C, C)])
        return carry

    lax.fori_loop(0, _nchunks(wid), body, 0)


@functools.lru_cache(maxsize=None)
def _attn():
    return pl.kernel(
        _attn_body,
        out_type=jax.ShapeDtypeStruct((E,), f32),
        mesh=_mesh(),
        scratch_types=[
            pltpu.VMEM((NP,), f32),
            pltpu.VMEM((NP,), f32),
            pltpu.VMEM((EPW,), jnp.int32),
            pltpu.VMEM((C,), f32),
            pltpu.VMEM((C,), f32),
        ],
        compiler_params=pltpu.CompilerParams(needs_layout_passes=False),
    )


def _k1_body(x_ref, w_ref, h_ref):
    h_ref[...] = jnp.dot(x_ref[...], w_ref[...], preferred_element_type=f32)


def _k1(x, w_t):
    nb = 1000
    return pl.pallas_call(
        _k1_body,
        grid=(N // nb,),
        in_specs=[pl.BlockSpec((nb, D), lambda i: (i, 0)),
                  pl.BlockSpec((D, D), lambda i: (0, 0))],
        out_specs=pl.BlockSpec((nb, D), lambda i: (i, 0)),
        out_shape=jax.ShapeDtypeStruct((N, D), f32),
    )(x, w_t)


_EB = 512


def _k3_body(hsrc_ref, eat_ref, we_ref, be_ref, av_ref, wmsg_ref, p_ref):
    eat = eat_ref[...]
    nrm = jnp.sqrt(jnp.sum(eat * eat, axis=0, keepdims=True)) + 1e-8
    ean_t = eat / nrm
    ep = lax.dot_general(ean_t, we_ref[...], (((0,), (1,)), ((), ())),
                         preferred_element_type=f32) + be_ref[...]
    msgs = jnp.tanh(hsrc_ref[...] + ep)
    att = jnp.dot(msgs, av_ref[...], preferred_element_type=f32)
    p = jnp.exp(att)
    wmsg_ref[...] = msgs * p
    att_row = lax.dot_general(av_ref[...], msgs, (((0,), (1,)), ((), ())),
                              preferred_element_type=f32)
    p_ref[...] = jnp.exp(att_row).reshape(1, 1, _EB)


def _k3(hsrc, eat, we, be, av):
    nblk = E // _EB
    return pl.pallas_call(
        _k3_body,
        grid=(nblk,),
        in_specs=[pl.BlockSpec((_EB, D), lambda i: (i, 0)),
                  pl.BlockSpec((DE, _EB), lambda i: (0, i)),
                  pl.BlockSpec((D, DE), lambda i: (0, 0)),
                  pl.BlockSpec((1, D), lambda i: (0, 0)),
                  pl.BlockSpec((D, 1), lambda i: (0, 0))],
        out_specs=[pl.BlockSpec((_EB, D), lambda i: (i, 0)),
                   pl.BlockSpec((1, 1, _EB), lambda i: (i, 0, 0))],
        out_shape=(jax.ShapeDtypeStruct((E, D), f32),
                   jax.ShapeDtypeStruct((nblk, 1, _EB), f32)),
    )(hsrc, eat, we, be, av)


_NB = 1000


def _k5a_body(h_ref, agg_ref, den0_ref, den1_ref, eps_ref, wu1_ref, bu1_ref,
              wu2_ref, bu2_ref, wo1_ref, bo1_ref,
              o1_ref, csum_ref, csq_ref):
    i = pl.program_id(0)
    den = (den0_ref[...].reshape(_NB, 1) + den1_ref[...].reshape(_NB, 1)
           + 1e-16)
    agg = (agg_ref[0] + agg_ref[1]) / den
    pre = (1.0 + eps_ref[0, 0]) * h_ref[...] + agg
    t = jnp.tanh(jnp.dot(pre, wu1_ref[...], preferred_element_type=f32)
                 + bu1_ref[...])
    u = jnp.dot(t, wu2_ref[...], preferred_element_type=f32) + bu2_ref[...]
    o1 = jnp.dot(u, wo1_ref[...], preferred_element_type=f32) + bo1_ref[...]
    o1_ref[...] = o1
    s = jnp.sum(o1, axis=0, keepdims=True)
    s2 = jnp.sum(o1 * o1, axis=0, keepdims=True)

    @pl.when(i == 0)
    def _():
        csum_ref[...] = s
        csq_ref[...] = s2

    @pl.when(i > 0)
    def _():
        csum_ref[...] = csum_ref[...] + s
        csq_ref[...] = csq_ref[...] + s2


def _k5a(h, aggpart, den0, den1, eps2, wu1t, bu1, wu2t, bu2, wo1t, bo1):
    nblk = N // _NB
    return pl.pallas_call(
        _k5a_body,
        grid=(nblk,),
        in_specs=[pl.BlockSpec((_NB, D), lambda i: (i, 0)),
                  pl.BlockSpec((NC, _NB, D), lambda i: (0, i, 0)),
                  pl.BlockSpec((1, _NB, 1), lambda i: (i, 0, 0)),
                  pl.BlockSpec((1, _NB, 1), lambda i: (i, 0, 0)),
                  pl.BlockSpec((1, 1), lambda i: (0, 0)),
                  pl.BlockSpec((D, HID), lambda i: (0, 0)),
                  pl.BlockSpec((1, HID), lambda i: (0, 0)),
                  pl.BlockSpec((HID, D), lambda i: (0, 0)),
                  pl.BlockSpec((1, D), lambda i: (0, 0)),
                  pl.BlockSpec((D, HID), lambda i: (0, 0)),
                  pl.BlockSpec((1, HID), lambda i: (0, 0))],
        out_specs=[pl.BlockSpec((_NB, HID), lambda i: (i, 0)),
                   pl.BlockSpec((1, HID), lambda i: (0, 0)),
                   pl.BlockSpec((1, HID), lambda i: (0, 0))],
        out_shape=(jax.ShapeDtypeStruct((N, HID), f32),
                   jax.ShapeDtypeStruct((1, HID), f32),
                   jax.ShapeDtypeStruct((1, HID), f32)),
    )(h, aggpart, den0, den1, eps2, wu1t, bu1, wu2t, bu2, wo1t, bo1)


def _k5b_body(o1_ref, csum_ref, csq_ref, g_ref, b_ref, wo2_ref, bo2_ref,
              out_ref):
    mean = csum_ref[...] * (1.0 / N)
    var = csq_ref[...] * (1.0 / N) - mean * mean
    rstd = 1.0 / jnp.sqrt(var + 1e-5)
    o1n = (o1_ref[...] - mean) * rstd * g_ref[...] + b_ref[...]
    out_ref[...] = (jnp.dot(jnp.tanh(o1n), wo2_ref[...],
                            preferred_element_type=f32) + bo2_ref[...])


def _k5b(o1, csum, csq, g, b, wo2t, bo2):
    nblk = N // _NB
    return pl.pallas_call(
        _k5b_body,
        grid=(nblk,),
        in_specs=[pl.BlockSpec((_NB, HID), lambda i: (i, 0)),
                  pl.BlockSpec((1, HID), lambda i: (0, 0)),
                  pl.BlockSpec((1, HID), lambda i: (0, 0)),
                  pl.BlockSpec((1, HID), lambda i: (0, 0)),
                  pl.BlockSpec((1, HID), lambda i: (0, 0)),
                  pl.BlockSpec((HID, D), lambda i: (0, 0)),
                  pl.BlockSpec((1, D), lambda i: (0, 0))],
        out_specs=pl.BlockSpec((_NB, D), lambda i: (i, 0)),
        out_shape=jax.ShapeDtypeStruct((N, D), f32),
    )(o1, csum, csq, g, b, wo2t, bo2)


def kernel(x, edge_index, edge_attr, W_init, W_edge, b_edge, eps, att_vec,
           W_u1, b_u1, W_u2, b_u2, W_o1, b_o1, bn_gamma, bn_beta, W_o2, b_o2):
    src = edge_index[0]
    dst = edge_index[1]
    srcp = jnp.pad(src, (0, E_PAD - E))
    dstf = jnp.pad(dst, (0, E_PAD - E))
    zeros = jnp.zeros((N, D), f32)

    h = _k1(x, W_init.T)
    hsrc = _gather()(h, srcp)
    wmsg, p3 = _k3(hsrc, edge_attr.T, W_edge, b_edge.reshape(1, D), att_vec)
    p = p3.reshape(E)
    aggpart, denpart = _scatter()(wmsg, p, dstf, zeros, jnp.zeros((NP,), f32))
    dp = denpart.reshape(NC, NP)
    den0 = dp[0, :N].reshape(N // _NB, _NB, 1)
    den1 = dp[1, :N].reshape(N // _NB, _NB, 1)
    o1, csum, csq = _k5a(h, aggpart, den0, den1, eps.reshape(1, 1),
                         W_u1.T, b_u1.reshape(1, HID),
                         W_u2.T, b_u2.reshape(1, D),
                         W_o1.T, b_o1.reshape(1, HID))
    out = _k5b(o1, csum, csq, bn_gamma.reshape(1, HID),
               bn_beta.reshape(1, HID), W_o2.T, b_o2.reshape(1, D))
    attn = _attn()(p, dstf, dp[0], dp[1])
    return out, attn


# speedup vs baseline: 11.7368x; 1.6027x over previous
import functools

import jax
import jax.numpy as jnp
from jax import lax
from jax.experimental import pallas as pl
from jax.experimental.pallas import tpu as pltpu
from jax.experimental.pallas import tpu_sc as plsc

N = 10000
E = 320000
D = 128
DE = 16
HID = 128

NC = 2
NS = 16
NW = NC * NS
C = 128
CPW = 79
EPW = CPW * C
E_PAD = NW * EPW
LAST_REAL = (E - (NW - 1) * EPW) // C
NP = 10240

f32 = jnp.float32


@functools.lru_cache(maxsize=None)
def _mesh():
    return plsc.VectorSubcoreMesh(core_axis_name="c", subcore_axis_name="s",
                                  num_cores=NC, num_subcores=NS)


def _wid_base():
    c = lax.axis_index("c")
    s = lax.axis_index("s")
    wid = s * NC + c
    return c, s, wid, wid * EPW


def _nchunks(wid):
    return jnp.where(wid == NW - 1, LAST_REAL, CPW)


def _gather_body(h_hbm, srcp_hbm, hsrc_hbm, idx_v, buf, sem0, sem1):
    _, _, wid, base = _wid_base()
    pltpu.sync_copy(srcp_hbm.at[pl.ds(wid * EPW, EPW)], idx_v)
    n = _nchunks(wid)
    sems = (sem0, sem1)

    def start(j, slot):
        pltpu.async_copy(h_hbm.at[idx_v.at[pl.ds(j * C, C)]],
                         buf.at[slot], sems[slot])

    def finish(j, slot):
        pltpu.make_async_copy(h_hbm.at[pl.ds(0, C)], buf.at[slot],
                              sems[slot]).wait()
        pltpu.sync_copy(buf.at[slot], hsrc_hbm.at[pl.ds(base + j * C, C)])

    start(0, 0)

    def body(i, carry):
        j0 = 2 * i
        j1 = j0 + 1
        start(j1, 1)
        finish(j0, 0)

        @pl.when(j1 + 1 < n)
        def _():
            start(j1 + 1, 0)

        finish(j1, 1)
        return carry

    lax.fori_loop(0, (n - 1) // 2, body, 0)
    finish(n - 1, 0)


@functools.lru_cache(maxsize=None)
def _gather():
    return pl.kernel(
        _gather_body,
        out_type=jax.ShapeDtypeStruct((E, D), f32),
        mesh=_mesh(),
        scratch_types=[
            pltpu.VMEM((EPW,), jnp.int32),
            pltpu.VMEM((2, C, D), f32),
            pltpu.SemaphoreType.DMA,
            pltpu.SemaphoreType.DMA,
        ],
    )


_AS = 632
_DS = NP // NS


def _scatter_body(wmsg_hbm, p_hbm, dstp_hbm, zeros_hbm, zf_hbm,
                  aggpart_hbm, denpart_hbm,
                  agg_sh, den_sh, idx_v, buf0, buf1, pbuf0, pbuf1,
                  semw0, semw1, semp0, semp1):
    c, s, wid, base = _wid_base()
    semw = (semw0, semw1)
    semp = (semp0, semp1)
    bufs = (buf0, buf1)
    pbufs = (pbuf0, pbuf1)

    def zagg(sz):
        pltpu.sync_copy(zeros_hbm.at[pl.ds(s * _AS, sz)],
                        agg_sh.at[pl.ds(s * _AS, sz)])

    @pl.when(s < NS - 1)
    def _():
        zagg(_AS)

    @pl.when(s == NS - 1)
    def _():
        zagg(N - (NS - 1) * _AS)

    pltpu.sync_copy(zf_hbm.at[pl.ds(s * _DS, _DS)],
                    den_sh.at[pl.ds(s * _DS, _DS)])
    plsc.subcore_barrier()

    pltpu.sync_copy(dstp_hbm.at[pl.ds(wid * EPW, EPW)], idx_v)
    n = _nchunks(wid)

    def start(j, slot):
        pltpu.async_copy(wmsg_hbm.at[pl.ds(base + j * C, C)],
                         bufs[slot], semw[slot])
        pltpu.async_copy(p_hbm.at[pl.ds(base + j * C, C)],
                         pbufs[slot], semp[slot])

    def finish(j, slot):
        pltpu.make_async_copy(wmsg_hbm.at[pl.ds(0, C)], bufs[slot],
                              semw[slot]).wait()
        pltpu.make_async_copy(p_hbm.at[pl.ds(0, C)], pbufs[slot],
                              semp[slot]).wait()
        pltpu.sync_copy(bufs[slot],
                        agg_sh.at[idx_v.at[pl.ds(j * C, C)]], add=True)
        pltpu.sync_copy(pbufs[slot],
                        den_sh.at[idx_v.at[pl.ds(j * C, C)]], add=True)

    start(0, 0)

    def body(i, carry):
        j0 = 2 * i
        j1 = j0 + 1
        start(j1, 1)
        finish(j0, 0)

        @pl.when(j1 + 1 < n)
        def _():
            start(j1 + 1, 0)

        finish(j1, 1)
        return carry

    lax.fori_loop(0, (n - 1) // 2, body, 0)
    finish(n - 1, 0)
    plsc.subcore_barrier()

    def wagg(sz):
        pltpu.sync_copy(agg_sh.at[pl.ds(s * _AS, sz)],
                        aggpart_hbm.at[c, pl.ds(s * _AS, sz)])

    @pl.when(s < NS - 1)
    def _():
        wagg(_AS)

    @pl.when(s == NS - 1)
    def _():
        wagg(N - (NS - 1) * _AS)

    pltpu.sync_copy(den_sh.at[pl.ds(s * _DS, _DS)],
                    denpart_hbm.at[pl.ds(c * NP + s * _DS, _DS)])


@functools.lru_cache(maxsize=None)
def _scatter():
    return pl.kernel(
        _scatter_body,
        out_type=(jax.ShapeDtypeStruct((NC, N, D), f32),
                  jax.ShapeDtypeStruct((NC * NP,), f32)),
        mesh=_mesh(),
        scratch_types=[
            pltpu.VMEM_SHARED((N, D), f32),
            pltpu.VMEM_SHARED((NP,), f32),
            pltpu.VMEM((EPW,), jnp.int32),
            pltpu.VMEM((C, D), f32),
            pltpu.VMEM((C, D), f32),
            pltpu.VMEM((C,), f32),
            pltpu.VMEM((C,), f32),
            pltpu.SemaphoreType.DMA,
            pltpu.SemaphoreType.DMA,
            pltpu.SemaphoreType.DMA,
            pltpu.SemaphoreType.DMA,
        ],
    )


def _attn_body(p_hbm, dstp_hbm, den0_hbm, den1_hbm, attn_hbm,
               den_v, dbb, idx_v, pbuf, abuf):
    _, _, wid, base = _wid_base()
    pltpu.sync_copy(den0_hbm, den_v)
    pltpu.sync_copy(den1_hbm, dbb)

    def addden(i, carry):
        sl = pl.ds(i * 16, 16)
        den_v[sl] = den_v[sl] + dbb[sl] + 1e-16
        return carry

    lax.fori_loop(0, NP // 16, addden, 0)
    pltpu.sync_copy(dstp_hbm.at[pl.ds(wid * EPW, EPW)], idx_v)

    def body(j, carry):
        pltpu.sync_copy(p_hbm.at[pl.ds(base + j * C, C)], pbuf)
        for k in range(C // 16):
            idxk = idx_v[pl.ds(j * C + k * 16, 16)]
            d = plsc.load_gather(den_v, [idxk])
            abuf[pl.ds(k * 16, 16)] = pbuf[pl.ds(k * 16, 16)] / d
        pltpu.sync_copy(abuf, attn_hbm.at[pl.ds(base + j * C, C)])
        return carry

    lax.fori_loop(0, _nchunks(wid), body, 0)


@functools.lru_cache(maxsize=None)
def _attn():
    return pl.kernel(
        _attn_body,
        out_type=jax.ShapeDtypeStruct((E,), f32),
        mesh=_mesh(),
        scratch_types=[
            pltpu.VMEM((NP,), f32),
            pltpu.VMEM((NP,), f32),
            pltpu.VMEM((EPW,), jnp.int32),
            pltpu.VMEM((C,), f32),
            pltpu.VMEM((C,), f32),
        ],
        compiler_params=pltpu.CompilerParams(needs_layout_passes=False),
    )


def _k1_body(x_ref, w_ref, h_ref):
    h_ref[...] = jnp.dot(x_ref[...], w_ref[...], preferred_element_type=f32)


def _k1(x, w_t):
    nb = 1000
    return pl.pallas_call(
        _k1_body,
        grid=(N // nb,),
        in_specs=[pl.BlockSpec((nb, D), lambda i: (i, 0)),
                  pl.BlockSpec((D, D), lambda i: (0, 0))],
        out_specs=pl.BlockSpec((nb, D), lambda i: (i, 0)),
        out_shape=jax.ShapeDtypeStruct((N, D), f32),
    )(x, w_t)


_EB = 2560


def _k3_body(hsrc_ref, eat_ref, we_ref, be_ref, av_ref, wmsg_ref, p_ref):
    eat = eat_ref[...]
    nrm = jnp.sqrt(jnp.sum(eat * eat, axis=0, keepdims=True)) + 1e-8
    ean_t = eat / nrm
    ep = lax.dot_general(ean_t, we_ref[...], (((0,), (1,)), ((), ())),
                         preferred_element_type=f32) + be_ref[...]
    msgs = jnp.tanh(hsrc_ref[...] + ep)
    att = jnp.dot(msgs, av_ref[...], preferred_element_type=f32)
    p = jnp.exp(att)
    wmsg_ref[...] = msgs * p
    att_row = lax.dot_general(av_ref[...], msgs, (((0,), (1,)), ((), ())),
                              preferred_element_type=f32)
    p_ref[...] = jnp.exp(att_row).reshape(1, 1, _EB)


def _k3(hsrc, eat, we, be, av):
    nblk = E // _EB
    return pl.pallas_call(
        _k3_body,
        grid=(nblk,),
        in_specs=[pl.BlockSpec((_EB, D), lambda i: (i, 0)),
                  pl.BlockSpec((DE, _EB), lambda i: (0, i)),
                  pl.BlockSpec((D, DE), lambda i: (0, 0)),
                  pl.BlockSpec((1, D), lambda i: (0, 0)),
                  pl.BlockSpec((D, 1), lambda i: (0, 0))],
        out_specs=[pl.BlockSpec((_EB, D), lambda i: (i, 0)),
                   pl.BlockSpec((1, 1, _EB), lambda i: (i, 0, 0))],
        out_shape=(jax.ShapeDtypeStruct((E, D), f32),
                   jax.ShapeDtypeStruct((nblk, 1, _EB), f32)),
    )(hsrc, eat, we, be, av)


_NB = 1000


def _k5a_body(h_ref, agg_ref, den0_ref, den1_ref, eps_ref, wu1_ref, bu1_ref,
              wu2_ref, bu2_ref, wo1_ref, bo1_ref,
              o1_ref, csum_ref, csq_ref):
    i = pl.program_id(0)
    den = (den0_ref[...].reshape(_NB, 1) + den1_ref[...].reshape(_NB, 1)
           + 1e-16)
    agg = (agg_ref[0] + agg_ref[1]) / den
    pre = (1.0 + eps_ref[0, 0]) * h_ref[...] + agg
    t = jnp.tanh(jnp.dot(pre, wu1_ref[...], preferred_element_type=f32)
                 + bu1_ref[...])
    u = jnp.dot(t, wu2_ref[...], preferred_element_type=f32) + bu2_ref[...]
    o1 = jnp.dot(u, wo1_ref[...], preferred_element_type=f32) + bo1_ref[...]
    o1_ref[...] = o1
    s = jnp.sum(o1, axis=0, keepdims=True)
    s2 = jnp.sum(o1 * o1, axis=0, keepdims=True)

    @pl.when(i == 0)
    def _():
        csum_ref[...] = s
        csq_ref[...] = s2

    @pl.when(i > 0)
    def _():
        csum_ref[...] = csum_ref[...] + s
        csq_ref[...] = csq_ref[...] + s2


def _k5a(h, aggpart, den0, den1, eps2, wu1t, bu1, wu2t, bu2, wo1t, bo1):
    nblk = N // _NB
    return pl.pallas_call(
        _k5a_body,
        grid=(nblk,),
        in_specs=[pl.BlockSpec((_NB, D), lambda i: (i, 0)),
                  pl.BlockSpec((NC, _NB, D), lambda i: (0, i, 0)),
                  pl.BlockSpec((1, _NB, 1), lambda i: (i, 0, 0)),
                  pl.BlockSpec((1, _NB, 1), lambda i: (i, 0, 0)),
                  pl.BlockSpec((1, 1), lambda i: (0, 0)),
                  pl.BlockSpec((D, HID), lambda i: (0, 0)),
                  pl.BlockSpec((1, HID), lambda i: (0, 0)),
                  pl.BlockSpec((HID, D), lambda i: (0, 0)),
                  pl.BlockSpec((1, D), lambda i: (0, 0)),
                  pl.BlockSpec((D, HID), lambda i: (0, 0)),
                  pl.BlockSpec((1, HID), lambda i: (0, 0))],
        out_specs=[pl.BlockSpec((_NB, HID), lambda i: (i, 0)),
                   pl.BlockSpec((1, HID), lambda i: (0, 0)),
                   pl.BlockSpec((1, HID), lambda i: (0, 0))],
        out_shape=(jax.ShapeDtypeStruct((N, HID), f32),
                   jax.ShapeDtypeStruct((1, HID), f32),
                   jax.ShapeDtypeStruct((1, HID), f32)),
    )(h, aggpart, den0, den1, eps2, wu1t, bu1, wu2t, bu2, wo1t, bo1)


def _k5b_body(o1_ref, csum_ref, csq_ref, g_ref, b_ref, wo2_ref, bo2_ref,
              out_ref):
    mean = csum_ref[...] * (1.0 / N)
    var = csq_ref[...] * (1.0 / N) - mean * mean
    rstd = 1.0 / jnp.sqrt(var + 1e-5)
    o1n = (o1_ref[...] - mean) * rstd * g_ref[...] + b_ref[...]
    out_ref[...] = (jnp.dot(jnp.tanh(o1n), wo2_ref[...],
                            preferred_element_type=f32) + bo2_ref[...])


def _k5b(o1, csum, csq, g, b, wo2t, bo2):
    nblk = N // _NB
    return pl.pallas_call(
        _k5b_body,
        grid=(nblk,),
        in_specs=[pl.BlockSpec((_NB, HID), lambda i: (i, 0)),
                  pl.BlockSpec((1, HID), lambda i: (0, 0)),
                  pl.BlockSpec((1, HID), lambda i: (0, 0)),
                  pl.BlockSpec((1, HID), lambda i: (0, 0)),
                  pl.BlockSpec((1, HID), lambda i: (0, 0)),
                  pl.BlockSpec((HID, D), lambda i: (0, 0)),
                  pl.BlockSpec((1, D), lambda i: (0, 0))],
        out_specs=pl.BlockSpec((_NB, D), lambda i: (i, 0)),
        out_shape=jax.ShapeDtypeStruct((N, D), f32),
    )(o1, csum, csq, g, b, wo2t, bo2)


def kernel(x, edge_index, edge_attr, W_init, W_edge, b_edge, eps, att_vec,
           W_u1, b_u1, W_u2, b_u2, W_o1, b_o1, bn_gamma, bn_beta, W_o2, b_o2):
    src = edge_index[0]
    dst = edge_index[1]
    srcp = jnp.pad(src, (0, E_PAD - E))
    dstf = jnp.pad(dst, (0, E_PAD - E))
    zeros = jnp.zeros((N, D), f32)

    h = _k1(x, W_init.T)
    hsrc = _gather()(h, srcp)
    wmsg, p3 = _k3(hsrc, edge_attr.T, W_edge, b_edge.reshape(1, D), att_vec)
    p = p3.reshape(E)
    aggpart, denpart = _scatter()(wmsg, p, dstf, zeros, jnp.zeros((NP,), f32))
    dp = denpart.reshape(NC, NP)
    den0 = dp[0, :N].reshape(N // _NB, _NB, 1)
    den1 = dp[1, :N].reshape(N // _NB, _NB, 1)
    o1, csum, csq = _k5a(h, aggpart, den0, den1, eps.reshape(1, 1),
                         W_u1.T, b_u1.reshape(1, HID),
                         W_u2.T, b_u2.reshape(1, D),
                         W_o1.T, b_o1.reshape(1, HID))
    out = _k5b(o1, csum, csq, bn_gamma.reshape(1, HID),
               bn_beta.reshape(1, HID), W_o2.T, b_o2.reshape(1, D))
    attn = _attn()(p, dstf, dp[0], dp[1])
    return out, attn


# speedup vs baseline: 12.1218x; 1.0328x over previous
import functools

import jax
import jax.numpy as jnp
from jax import lax
from jax.experimental import pallas as pl
from jax.experimental.pallas import tpu as pltpu
from jax.experimental.pallas import tpu_sc as plsc

N = 10000
E = 320000
D = 128
DE = 16
HID = 128

NC = 2
NS = 16
NW = NC * NS
C = 128
CPW = 79
EPW = CPW * C
E_PAD = NW * EPW
LAST_REAL = (E - (NW - 1) * EPW) // C
NP = 10240

f32 = jnp.float32


@functools.lru_cache(maxsize=None)
def _mesh():
    return plsc.VectorSubcoreMesh(core_axis_name="c", subcore_axis_name="s",
                                  num_cores=NC, num_subcores=NS)


def _wid_base():
    c = lax.axis_index("c")
    s = lax.axis_index("s")
    wid = s * NC + c
    return c, s, wid, wid * EPW


def _nchunks(wid):
    return jnp.where(wid == NW - 1, LAST_REAL, CPW)


def _gather_body(h_hbm, srcp_hbm, hsrc_hbm, idx_v, buf0, buf1, buf2, buf3,
                 sem0, sem1, sem2, sem3):
    _, _, wid, base = _wid_base()
    pltpu.sync_copy(srcp_hbm.at[pl.ds(wid * EPW, EPW)], idx_v)
    n = _nchunks(wid)
    sems = (sem0, sem1, sem2, sem3)
    bufs = (buf0, buf1, buf2, buf3)

    def start(j, slot):
        pltpu.async_copy(h_hbm.at[idx_v.at[pl.ds(j * C, C)]],
                         bufs[slot], sems[slot])

    def finish(j, slot):
        pltpu.make_async_copy(h_hbm.at[pl.ds(0, C)], bufs[slot],
                              sems[slot]).wait()
        pltpu.sync_copy(bufs[slot], hsrc_hbm.at[pl.ds(base + j * C, C)])

    start(0, 0)
    start(1, 1)
    start(2, 2)

    def body(i, carry):
        j = 4 * i
        start(j + 3, 3)
        finish(j, 0)

        @pl.when(j + 4 < n)
        def _():
            start(j + 4, 0)

        finish(j + 1, 1)

        @pl.when(j + 5 < n)
        def _():
            start(j + 5, 1)

        finish(j + 2, 2)

        @pl.when(j + 6 < n)
        def _():
            start(j + 6, 2)

        finish(j + 3, 3)
        return carry

    lax.fori_loop(0, (n - 3) // 4, body, 0)
    finish(n - 3, 0)
    finish(n - 2, 1)
    finish(n - 1, 2)


@functools.lru_cache(maxsize=None)
def _gather():
    return pl.kernel(
        _gather_body,
        out_type=jax.ShapeDtypeStruct((E, D), f32),
        mesh=_mesh(),
        scratch_types=[
            pltpu.VMEM((EPW,), jnp.int32),
            pltpu.VMEM((C, D), f32),
            pltpu.VMEM((C, D), f32),
            pltpu.VMEM((C, D), f32),
            pltpu.VMEM((C, D), f32),
            pltpu.SemaphoreType.DMA,
            pltpu.SemaphoreType.DMA,
            pltpu.SemaphoreType.DMA,
            pltpu.SemaphoreType.DMA,
        ],
    )


_AS = 632
_DS = NP // NS


def _scatter_body(wmsg_hbm, p_hbm, dstp_hbm, zeros_hbm, zf_hbm,
                  aggpart_hbm, denpart_hbm,
                  agg_sh, den_sh, idx_v, buf0, buf1, pbuf0, pbuf1,
                  semw0, semw1, semp0, semp1):
    c, s, wid, base = _wid_base()
    semw = (semw0, semw1)
    semp = (semp0, semp1)
    bufs = (buf0, buf1)
    pbufs = (pbuf0, pbuf1)

    def zagg(sz):
        pltpu.sync_copy(zeros_hbm.at[pl.ds(s * _AS, sz)],
                        agg_sh.at[pl.ds(s * _AS, sz)])

    @pl.when(s < NS - 1)
    def _():
        zagg(_AS)

    @pl.when(s == NS - 1)
    def _():
        zagg(N - (NS - 1) * _AS)

    pltpu.sync_copy(zf_hbm.at[pl.ds(s * _DS, _DS)],
                    den_sh.at[pl.ds(s * _DS, _DS)])
    plsc.subcore_barrier()

    pltpu.sync_copy(dstp_hbm.at[pl.ds(wid * EPW, EPW)], idx_v)
    n = _nchunks(wid)

    def start(j, slot):
        pltpu.async_copy(wmsg_hbm.at[pl.ds(base + j * C, C)],
                         bufs[slot], semw[slot])
        pltpu.async_copy(p_hbm.at[pl.ds(base + j * C, C)],
                         pbufs[slot], semp[slot])

    def finish(j, slot):
        pltpu.make_async_copy(wmsg_hbm.at[pl.ds(0, C)], bufs[slot],
                              semw[slot]).wait()
        pltpu.make_async_copy(p_hbm.at[pl.ds(0, C)], pbufs[slot],
                              semp[slot]).wait()
        pltpu.sync_copy(bufs[slot],
                        agg_sh.at[idx_v.at[pl.ds(j * C, C)]], add=True)
        pltpu.sync_copy(pbufs[slot],
                        den_sh.at[idx_v.at[pl.ds(j * C, C)]], add=True)

    start(0, 0)

    def body(i, carry):
        j0 = 2 * i
        j1 = j0 + 1
        start(j1, 1)
        finish(j0, 0)

        @pl.when(j1 + 1 < n)
        def _():
            start(j1 + 1, 0)

        finish(j1, 1)
        return carry

    lax.fori_loop(0, (n - 1) // 2, body, 0)
    finish(n - 1, 0)
    plsc.subcore_barrier()

    def wagg(sz):
        pltpu.sync_copy(agg_sh.at[pl.ds(s * _AS, sz)],
                        aggpart_hbm.at[c, pl.ds(s * _AS, sz)])

    @pl.when(s < NS - 1)
    def _():
        wagg(_AS)

    @pl.when(s == NS - 1)
    def _():
        wagg(N - (NS - 1) * _AS)

    pltpu.sync_copy(den_sh.at[pl.ds(s * _DS, _DS)],
                    denpart_hbm.at[pl.ds(c * NP + s * _DS, _DS)])


@functools.lru_cache(maxsize=None)
def _scatter():
    return pl.kernel(
        _scatter_body,
        out_type=(jax.ShapeDtypeStruct((NC, N, D), f32),
                  jax.ShapeDtypeStruct((NC * NP,), f32)),
        mesh=_mesh(),
        scratch_types=[
            pltpu.VMEM_SHARED((N, D), f32),
            pltpu.VMEM_SHARED((NP,), f32),
            pltpu.VMEM((EPW,), jnp.int32),
            pltpu.VMEM((C, D), f32),
            pltpu.VMEM((C, D), f32),
            pltpu.VMEM((C,), f32),
            pltpu.VMEM((C,), f32),
            pltpu.SemaphoreType.DMA,
            pltpu.SemaphoreType.DMA,
            pltpu.SemaphoreType.DMA,
            pltpu.SemaphoreType.DMA,
        ],
    )


def _attn_body(p_hbm, dstp_hbm, den0_hbm, den1_hbm, attn_hbm,
               den_v, dbb, idx_v, pbuf, abuf):
    _, _, wid, base = _wid_base()
    pltpu.sync_copy(den0_hbm, den_v)
    pltpu.sync_copy(den1_hbm, dbb)

    def addden(i, carry):
        sl = pl.ds(i * 16, 16)
        den_v[sl] = den_v[sl] + dbb[sl] + 1e-16
        return carry

    lax.fori_loop(0, NP // 16, addden, 0)
    pltpu.sync_copy(dstp_hbm.at[pl.ds(wid * EPW, EPW)], idx_v)

    def body(j, carry):
        pltpu.sync_copy(p_hbm.at[pl.ds(base + j * C, C)], pbuf)
        for k in range(C // 16):
            idxk = idx_v[pl.ds(j * C + k * 16, 16)]
            d = plsc.load_gather(den_v, [idxk])
            abuf[pl.ds(k * 16, 16)] = pbuf[pl.ds(k * 16, 16)] / d
        pltpu.sync_copy(abuf, attn_hbm.at[pl.ds(base + j * C, C)])
        return carry

    lax.fori_loop(0, _nchunks(wid), body, 0)


@functools.lru_cache(maxsize=None)
def _attn():
    return pl.kernel(
        _attn_body,
        out_type=jax.ShapeDtypeStruct((E,), f32),
        mesh=_mesh(),
        scratch_types=[
            pltpu.VMEM((NP,), f32),
            pltpu.VMEM((NP,), f32),
            pltpu.VMEM((EPW,), jnp.int32),
            pltpu.VMEM((C,), f32),
            pltpu.VMEM((C,), f32),
        ],
        compiler_params=pltpu.CompilerParams(needs_layout_passes=False),
    )


def _k1_body(x_ref, w_ref, h_ref):
    h_ref[...] = jnp.dot(x_ref[...], w_ref[...], preferred_element_type=f32)


def _k1(x, w_t):
    nb = 1000
    return pl.pallas_call(
        _k1_body,
        grid=(N // nb,),
        in_specs=[pl.BlockSpec((nb, D), lambda i: (i, 0)),
                  pl.BlockSpec((D, D), lambda i: (0, 0))],
        out_specs=pl.BlockSpec((nb, D), lambda i: (i, 0)),
        out_shape=jax.ShapeDtypeStruct((N, D), f32),
    )(x, w_t)


_EB = 3200


def _k3_body(hsrc_ref, eat_ref, we_ref, be_ref, av_ref, wmsg_ref, p_ref):
    eat = eat_ref[...]
    nrm = jnp.sqrt(jnp.sum(eat * eat, axis=0, keepdims=True)) + 1e-8
    ean_t = eat / nrm
    ep = lax.dot_general(ean_t, we_ref[...], (((0,), (1,)), ((), ())),
                         preferred_element_type=f32) + be_ref[...]
    msgs = jnp.tanh(hsrc_ref[...] + ep)
    att = jnp.dot(msgs, av_ref[...], preferred_element_type=f32)
    p = jnp.exp(att)
    wmsg_ref[...] = msgs * p
    att_row = lax.dot_general(av_ref[...], msgs, (((0,), (1,)), ((), ())),
                              preferred_element_type=f32)
    p_ref[...] = jnp.exp(att_row).reshape(1, 1, _EB)


def _k3(hsrc, eat, we, be, av):
    nblk = E // _EB
    return pl.pallas_call(
        _k3_body,
        grid=(nblk,),
        in_specs=[pl.BlockSpec((_EB, D), lambda i: (i, 0)),
                  pl.BlockSpec((DE, _EB), lambda i: (0, i)),
                  pl.BlockSpec((D, DE), lambda i: (0, 0)),
                  pl.BlockSpec((1, D), lambda i: (0, 0)),
                  pl.BlockSpec((D, 1), lambda i: (0, 0))],
        out_specs=[pl.BlockSpec((_EB, D), lambda i: (i, 0)),
                   pl.BlockSpec((1, 1, _EB), lambda i: (i, 0, 0))],
        out_shape=(jax.ShapeDtypeStruct((E, D), f32),
                   jax.ShapeDtypeStruct((nblk, 1, _EB), f32)),
    )(hsrc, eat, we, be, av)


_NB = 1000


def _k5a_body(h_ref, agg_ref, den0_ref, den1_ref, eps_ref, wu1_ref, bu1_ref,
              wu2_ref, bu2_ref, wo1_ref, bo1_ref,
              o1_ref, csum_ref, csq_ref):
    i = pl.program_id(0)
    den = (den0_ref[...].reshape(_NB, 1) + den1_ref[...].reshape(_NB, 1)
           + 1e-16)
    agg = (agg_ref[0] + agg_ref[1]) / den
    pre = (1.0 + eps_ref[0, 0]) * h_ref[...] + agg
    t = jnp.tanh(jnp.dot(pre, wu1_ref[...], preferred_element_type=f32)
                 + bu1_ref[...])
    u = jnp.dot(t, wu2_ref[...], preferred_element_type=f32) + bu2_ref[...]
    o1 = jnp.dot(u, wo1_ref[...], preferred_element_type=f32) + bo1_ref[...]
    o1_ref[...] = o1
    s = jnp.sum(o1, axis=0, keepdims=True)
    s2 = jnp.sum(o1 * o1, axis=0, keepdims=True)

    @pl.when(i == 0)
    def _():
        csum_ref[...] = s
        csq_ref[...] = s2

    @pl.when(i > 0)
    def _():
        csum_ref[...] = csum_ref[...] + s
        csq_ref[...] = csq_ref[...] + s2


def _k5a(h, aggpart, den0, den1, eps2, wu1t, bu1, wu2t, bu2, wo1t, bo1):
    nblk = N // _NB
    return pl.pallas_call(
        _k5a_body,
        grid=(nblk,),
        in_specs=[pl.BlockSpec((_NB, D), lambda i: (i, 0)),
                  pl.BlockSpec((NC, _NB, D), lambda i: (0, i, 0)),
                  pl.BlockSpec((1, _NB, 1), lambda i: (i, 0, 0)),
                  pl.BlockSpec((1, _NB, 1), lambda i: (i, 0, 0)),
                  pl.BlockSpec((1, 1), lambda i: (0, 0)),
                  pl.BlockSpec((D, HID), lambda i: (0, 0)),
                  pl.BlockSpec((1, HID), lambda i: (0, 0)),
                  pl.BlockSpec((HID, D), lambda i: (0, 0)),
                  pl.BlockSpec((1, D), lambda i: (0, 0)),
                  pl.BlockSpec((D, HID), lambda i: (0, 0)),
                  pl.BlockSpec((1, HID), lambda i: (0, 0))],
        out_specs=[pl.BlockSpec((_NB, HID), lambda i: (i, 0)),
                   pl.BlockSpec((1, HID), lambda i: (0, 0)),
                   pl.BlockSpec((1, HID), lambda i: (0, 0))],
        out_shape=(jax.ShapeDtypeStruct((N, HID), f32),
                   jax.ShapeDtypeStruct((1, HID), f32),
                   jax.ShapeDtypeStruct((1, HID), f32)),
    )(h, aggpart, den0, den1, eps2, wu1t, bu1, wu2t, bu2, wo1t, bo1)


def _k5b_body(o1_ref, csum_ref, csq_ref, g_ref, b_ref, wo2_ref, bo2_ref,
              out_ref):
    mean = csum_ref[...] * (1.0 / N)
    var = csq_ref[...] * (1.0 / N) - mean * mean
    rstd = 1.0 / jnp.sqrt(var + 1e-5)
    o1n = (o1_ref[...] - mean) * rstd * g_ref[...] + b_ref[...]
    out_ref[...] = (jnp.dot(jnp.tanh(o1n), wo2_ref[...],
                            preferred_element_type=f32) + bo2_ref[...])


def _k5b(o1, csum, csq, g, b, wo2t, bo2):
    nblk = N // _NB
    return pl.pallas_call(
        _k5b_body,
        grid=(nblk,),
        in_specs=[pl.BlockSpec((_NB, HID), lambda i: (i, 0)),
                  pl.BlockSpec((1, HID), lambda i: (0, 0)),
                  pl.BlockSpec((1, HID), lambda i: (0, 0)),
                  pl.BlockSpec((1, HID), lambda i: (0, 0)),
                  pl.BlockSpec((1, HID), lambda i: (0, 0)),
                  pl.BlockSpec((HID, D), lambda i: (0, 0)),
                  pl.BlockSpec((1, D), lambda i: (0, 0))],
        out_specs=pl.BlockSpec((_NB, D), lambda i: (i, 0)),
        out_shape=jax.ShapeDtypeStruct((N, D), f32),
    )(o1, csum, csq, g, b, wo2t, bo2)


def kernel(x, edge_index, edge_attr, W_init, W_edge, b_edge, eps, att_vec,
           W_u1, b_u1, W_u2, b_u2, W_o1, b_o1, bn_gamma, bn_beta, W_o2, b_o2):
    src = edge_index[0]
    dst = edge_index[1]
    srcp = jnp.pad(src, (0, E_PAD - E))
    dstf = jnp.pad(dst, (0, E_PAD - E))
    zeros = jnp.zeros((N, D), f32)

    h = _k1(x, W_init.T)
    hsrc = _gather()(h, srcp)
    wmsg, p3 = _k3(hsrc, edge_attr.T, W_edge, b_edge.reshape(1, D), att_vec)
    p = p3.reshape(E)
    aggpart, denpart = _scatter()(wmsg, p, dstf, zeros, jnp.zeros((NP,), f32))
    dp = denpart.reshape(NC, NP)
    den0 = dp[0, :N].reshape(N // _NB, _NB, 1)
    den1 = dp[1, :N].reshape(N // _NB, _NB, 1)
    o1, csum, csq = _k5a(h, aggpart, den0, den1, eps.reshape(1, 1),
                         W_u1.T, b_u1.reshape(1, HID),
                         W_u2.T, b_u2.reshape(1, D),
                         W_o1.T, b_o1.reshape(1, HID))
    out = _k5b(o1, csum, csq, bn_gamma.reshape(1, HID),
               bn_beta.reshape(1, HID), W_o2.T, b_o2.reshape(1, D))
    attn = _attn()(p, dstf, dp[0], dp[1])
    return out, attn
